# Initial kernel scaffold; baseline (speedup 1.0000x reference)
import jax
import jax.numpy as jnp
from jax.experimental import pallas as pl


def kernel(h, pos, edge_index, edge_attr, W_e1, b_e1, W_e2, b_e2, W_n1, b_n1, W_n2, b_n2, W_c1, b_c1, W_c2):
    raise NotImplementedError("write your pallas kernel here")


# speedup vs baseline: 2.1871x; 2.1871x over previous
import functools

import jax
import jax.numpy as jnp
from jax import lax
from jax.experimental import pallas as pl
from jax.experimental.pallas import tpu as pltpu
from jax.experimental.pallas import tpu_sc as plsc

N, E, D, DE = 10000, 320000, 128, 16
DP = 144
NC, NS = 2, 16
NW = NC * NS

GCH = 80
GB_PER_W = (2 * E) // NW
GCHUNKS = GB_PER_W // GCH

def _vector_mesh():
    return plsc.VectorSubcoreMesh(core_axis_name="c", subcore_axis_name="s")


_SC_PARAMS = pltpu.CompilerParams(use_tc_tiling_on_sc=False)


@jax.jit
def _sc_gather(table, idx2):
    @functools.partial(
        pl.kernel,
        mesh=_vector_mesh(),
        compiler_params=_SC_PARAMS,
        out_type=jax.ShapeDtypeStruct((2 * E, DP), jnp.float32),
        scratch_types=[
            pltpu.VMEM((GCH,), jnp.int32),
            pltpu.VMEM((GCH, DP), jnp.float32),
            pltpu.SemaphoreType.DMA,
        ],
    )
    def gk(table_hbm, idx_hbm, out_hbm, idx_v, rows_v, sem):
        wid = lax.axis_index("c") * NS + lax.axis_index("s")
        base = wid * GB_PER_W

        @pl.loop(0, GCHUNKS)
        def _(ch):
            off = base + ch * GCH
            pltpu.sync_copy(idx_hbm.at[pl.ds(off, GCH)], idx_v)
            pltpu.async_copy(table_hbm.at[idx_v], rows_v, sem).wait()
            pltpu.sync_copy(rows_v, out_hbm.at[pl.ds(off, GCH)])

    return gk(table, idx2)


SCH = 80
SE_PER_W = E // NW
SCHUNKS = SE_PER_W // SCH
ZCHUNKS = N // SCH


@jax.jit
def _sc_scatter(edgeout, row):
    @functools.partial(
        pl.kernel,
        mesh=_vector_mesh(),
        compiler_params=_SC_PARAMS,
        out_type=jax.ShapeDtypeStruct((NC, N, DP), jnp.float32),
        scratch_types=[
            pltpu.VMEM((SCH,), jnp.int32),
            pltpu.VMEM((SCH, DP), jnp.float32),
            pltpu.VMEM_SHARED((N, DP), jnp.float32),
            pltpu.SemaphoreType.DMA,
        ],
    )
    def sk(vals_hbm, idx_hbm, out_hbm, idx_v, vals_v, acc_shared, sem):
        c = lax.axis_index("c")
        s = lax.axis_index("s")

        zero16 = jnp.zeros((16,), jnp.float32)

        @pl.loop(0, SCH)
        def _(r):
            @pl.loop(0, DP // 16)
            def _(cc):
                vals_v[r, pl.ds(cc * 16, 16)] = zero16

        @pl.loop(0, ZCHUNKS)
        def _(z):
            @pl.when(lax.rem(z, NS) == s)
            def _():
                pltpu.sync_copy(vals_v, acc_shared.at[pl.ds(z * SCH, SCH)])

        plsc.subcore_barrier()

        base = (c * NS + s) * SE_PER_W

        @pl.loop(0, SCHUNKS)
        def _(ch):
            off = base + ch * SCH
            pltpu.sync_copy(idx_hbm.at[pl.ds(off, SCH)], idx_v)
            pltpu.sync_copy(vals_hbm.at[pl.ds(off, SCH)], vals_v)
            pltpu.sync_copy(vals_v, acc_shared.at[idx_v], add=True)

        plsc.subcore_barrier()

        @pl.loop(0, ZCHUNKS)
        def _(z):
            @pl.when(lax.rem(z, NS) == s)
            def _():
                pltpu.sync_copy(acc_shared.at[pl.ds(z * SCH, SCH)],
                                out_hbm.at[c, pl.ds(z * SCH, SCH)])

    return sk(edgeout, row)


BE = 2000


def _silu(x):
    return x * jax.nn.sigmoid(x)


def _edge_body(gr_ref, gc_ref, ea_ref, w1r_ref, w1c_ref, w1a_ref, w1rad_ref,
               b1_ref, w2_ref, b2_ref, wc1_ref, bc1_ref, wc2_ref, out_ref):
    gr = gr_ref[...]
    gc = gc_ref[...]
    cd = gr[:, D:D + 3] - gc[:, D:D + 3]
    radial = jnp.sum(cd * cd, axis=1, keepdims=True)
    t1 = (jnp.dot(gr, w1r_ref[...], preferred_element_type=jnp.float32)
          + jnp.dot(gc, w1c_ref[...], preferred_element_type=jnp.float32)
          + jnp.dot(ea_ref[...], w1a_ref[...], preferred_element_type=jnp.float32)
          + radial * w1rad_ref[...]
          + b1_ref[...])
    x = _silu(t1)
    m = _silu(jnp.dot(x, w2_ref[...], preferred_element_type=jnp.float32)
              + b2_ref[...])
    c1 = _silu(jnp.dot(m, wc1_ref[...], preferred_element_type=jnp.float32)
               + bc1_ref[...])
    w = jnp.sum(c1 * wc2_ref[...], axis=1, keepdims=True)
    cu = cd * (w * lax.rsqrt(radial + 1e-8))
    ones = jnp.ones((BE, 1), jnp.float32)
    pad = jnp.zeros((BE, DP - D - 4), jnp.float32)
    out_ref[...] = jnp.concatenate([m, cu, ones, pad], axis=1)


def _edge_mlp(gathered, edge_attr, w1r, w1c, w1a, w1rad, b1, w2, b2, wc1, bc1, wc2):
    nb = E // BE
    full = lambda shape: pl.BlockSpec(shape, lambda i: (0, 0))
    return pl.pallas_call(
        _edge_body,
        grid=(nb,),
        in_specs=[
            pl.BlockSpec((BE, DP), lambda i: (i, 0)),
            pl.BlockSpec((BE, DP), lambda i: (i + nb, 0)),
            pl.BlockSpec((BE, DE), lambda i: (i, 0)),
            full((DP, D)), full((DP, D)), full((DE, D)), full((1, D)),
            full((1, D)), full((D, D)), full((1, D)),
            full((D, D)), full((1, D)), full((1, D)),
        ],
        out_specs=pl.BlockSpec((BE, DP), lambda i: (i, 0)),
        out_shape=jax.ShapeDtypeStruct((E, DP), jnp.float32),
    )(gathered, gathered, edge_attr, w1r, w1c, w1a, w1rad, b1, w2, b2, wc1, bc1, wc2)


BN = 2000


def _node_body(p0_ref, p1_ref, h_ref, pos_ref, wn1h_ref, wn1m_ref, bn1_ref,
               wn2_ref, bn2_ref, hnew_ref, posnew_ref):
    acc = p0_ref[...] + p1_ref[...]
    m_i = acc[:, :D]
    pu = acc[:, D:D + 3]
    cnt = acc[:, D + 3:D + 4]
    h = h_ref[...]
    t = _silu(jnp.dot(h, wn1h_ref[...], preferred_element_type=jnp.float32)
              + jnp.dot(m_i, wn1m_ref[...], preferred_element_type=jnp.float32)
              + bn1_ref[...])
    hnew_ref[...] = h + jnp.dot(t, wn2_ref[...], preferred_element_type=jnp.float32) + bn2_ref[...]
    posnew_ref[...] = pos_ref[...] + pu / (cnt + 1e-6)


def _node_mlp(p0, p1, h, pos, wn1h, wn1m, bn1, wn2, bn2):
    nb = N // BN
    full = lambda shape: pl.BlockSpec(shape, lambda i: (0, 0))
    return pl.pallas_call(
        _node_body,
        grid=(nb,),
        in_specs=[
            pl.BlockSpec((BN, DP), lambda i: (i, 0)),
            pl.BlockSpec((BN, DP), lambda i: (i, 0)),
            pl.BlockSpec((BN, D), lambda i: (i, 0)),
            pl.BlockSpec((BN, 3), lambda i: (i, 0)),
            full((D, D)), full((D, D)), full((1, D)),
            full((D, D)), full((1, D)),
        ],
        out_specs=[
            pl.BlockSpec((BN, D), lambda i: (i, 0)),
            pl.BlockSpec((BN, 3), lambda i: (i, 0)),
        ],
        out_shape=[
            jax.ShapeDtypeStruct((N, D), jnp.float32),
            jax.ShapeDtypeStruct((N, 3), jnp.float32),
        ],
    )(p0, p1, h, pos, wn1h, wn1m, bn1, wn2, bn2)


def kernel(h, pos, edge_index, edge_attr, W_e1, b_e1, W_e2, b_e2,
           W_n1, b_n1, W_n2, b_n2, W_c1, b_c1, W_c2):
    row, col = edge_index[0], edge_index[1]
    table = jnp.concatenate(
        [h, pos, jnp.zeros((N, DP - D - 3), jnp.float32)], axis=1)
    idx2 = jnp.concatenate([row, col])

    gathered = _sc_gather(table, idx2)

    w1r = jnp.concatenate([W_e1[:D], jnp.zeros((DP - D, D), jnp.float32)])
    w1c = jnp.concatenate([W_e1[D:2 * D], jnp.zeros((DP - D, D), jnp.float32)])
    w1rad = W_e1[2 * D:2 * D + 1]
    w1a = W_e1[2 * D + 1:]
    edgeout = _edge_mlp(gathered, edge_attr, w1r, w1c, w1a, w1rad,
                        b_e1.reshape(1, D), W_e2, b_e2.reshape(1, D),
                        W_c1, b_c1.reshape(1, D), W_c2.reshape(1, D))

    partials = _sc_scatter(edgeout, row)

    h_new, pos_new = _node_mlp(partials[0], partials[1], h, pos,
                               W_n1[:D], W_n1[D:], b_n1.reshape(1, D),
                               W_n2, b_n2.reshape(1, D))
    return (h_new, pos_new)


# speedup vs baseline: 2.2034x; 1.0075x over previous
import functools

import jax
import jax.numpy as jnp
from jax import lax
from jax.experimental import pallas as pl
from jax.experimental.pallas import tpu as pltpu
from jax.experimental.pallas import tpu_sc as plsc

N, E, D, DE = 10000, 320000, 128, 16
DP = 256
CUN = 320
NC, NS = 2, 16
NW = NC * NS

GCH = 80
GB_PER_W = (2 * E) // NW
GCHUNKS = GB_PER_W // GCH

SCH = 80
SE_PER_W = E // NW
SCHUNKS = SE_PER_W // SCH
ZCHUNKS = N // SCH
CUCHUNKS = CUN // SCH


def _vector_mesh():
    return plsc.VectorSubcoreMesh(core_axis_name="c", subcore_axis_name="s")


@jax.jit
def _sc_gather(table, idx2):
    @functools.partial(
        pl.kernel,
        mesh=_vector_mesh(),
        out_type=jax.ShapeDtypeStruct((2 * E, DP), jnp.float32),
        scratch_types=[
            pltpu.VMEM((GCH,), jnp.int32),
            pltpu.VMEM((GCH, DP), jnp.float32),
            pltpu.SemaphoreType.DMA,
        ],
    )
    def gk(table_hbm, idx_hbm, out_hbm, idx_v, rows_v, sem):
        wid = lax.axis_index("c") * NS + lax.axis_index("s")
        base = wid * GB_PER_W

        @pl.loop(0, GCHUNKS)
        def _(ch):
            off = base + ch * GCH
            pltpu.sync_copy(idx_hbm.at[pl.ds(off, GCH)], idx_v)
            pltpu.async_copy(table_hbm.at[idx_v], rows_v, sem).wait()
            pltpu.sync_copy(rows_v, out_hbm.at[pl.ds(off, GCH)])

    return gk(table, idx2)


@jax.jit
def _sc_scatter(edgeout, row, cuidx):
    @functools.partial(
        pl.kernel,
        mesh=_vector_mesh(),
        out_type=[
            jax.ShapeDtypeStruct((NC, N, D), jnp.float32),
            jax.ShapeDtypeStruct((NC, CUN, D), jnp.float32),
        ],
        scratch_types=[
            pltpu.VMEM((SCH,), jnp.int32),
            pltpu.VMEM((SCH,), jnp.int32),
            pltpu.VMEM((SCH, D), jnp.float32),
            pltpu.VMEM((SCH, D), jnp.float32),
            pltpu.VMEM_SHARED((N, D), jnp.float32),
            pltpu.VMEM_SHARED((CUN, D), jnp.float32),
            pltpu.SemaphoreType.DMA,
        ],
    )
    def sk(vals_hbm, idx_hbm, cuidx_hbm, outm_hbm, outcu_hbm,
           idx_v, cuidx_v, mv, cv, macc, cuacc, sem):
        c = lax.axis_index("c")
        s = lax.axis_index("s")

        zero16 = jnp.zeros((16,), jnp.float32)

        @pl.loop(0, SCH)
        def _(r):
            @pl.loop(0, D // 16)
            def _(cc):
                mv[r, pl.ds(cc * 16, 16)] = zero16

        @pl.loop(0, ZCHUNKS)
        def _(z):
            @pl.when(lax.rem(z, NS) == s)
            def _():
                pltpu.sync_copy(mv, macc.at[pl.ds(z * SCH, SCH)])

        @pl.loop(0, CUCHUNKS)
        def _(z):
            @pl.when(z == s)
            def _():
                pltpu.sync_copy(mv, cuacc.at[pl.ds(z * SCH, SCH)])

        plsc.subcore_barrier()

        base = (c * NS + s) * SE_PER_W

        @pl.loop(0, SCHUNKS)
        def _(ch):
            off = base + ch * SCH
            pltpu.sync_copy(idx_hbm.at[pl.ds(off, SCH)], idx_v)
            pltpu.sync_copy(cuidx_hbm.at[pl.ds(off, SCH)], cuidx_v)
            pltpu.sync_copy(vals_hbm.at[pl.ds(off, SCH), pl.ds(0, D)], mv)
            pltpu.sync_copy(vals_hbm.at[pl.ds(off, SCH), pl.ds(D, D)], cv)
            pltpu.sync_copy(mv, macc.at[idx_v], add=True)
            pltpu.sync_copy(cv, cuacc.at[cuidx_v], add=True)

        plsc.subcore_barrier()

        @pl.loop(0, ZCHUNKS)
        def _(z):
            @pl.when(lax.rem(z, NS) == s)
            def _():
                pltpu.sync_copy(macc.at[pl.ds(z * SCH, SCH)],
                                outm_hbm.at[c, pl.ds(z * SCH, SCH)])

        @pl.loop(0, CUCHUNKS)
        def _(z):
            @pl.when(z == s)
            def _():
                pltpu.sync_copy(cuacc.at[pl.ds(z * SCH, SCH)],
                                outcu_hbm.at[c, pl.ds(z * SCH, SCH)])

    return sk(edgeout, row, cuidx)


BE = 2560


def _silu(x):
    return x * jax.nn.sigmoid(x)


def _edge_body(gr_ref, gc_ref, ea_ref, rm_ref, w1r_ref, w1c_ref, w1a_ref,
               w1rad_ref, b1_ref, w2_ref, b2_ref, wc1_ref, bc1_ref, wc2_ref,
               out_ref):
    gr = gr_ref[...]
    gc = gc_ref[...]
    hr = gr[:, :D]
    hc = gc[:, :D]
    cd = gr[:, D:D + 3] - gc[:, D:D + 3]
    radial = jnp.sum(cd * cd, axis=1, keepdims=True)
    t1 = (jnp.dot(hr, w1r_ref[...], preferred_element_type=jnp.float32)
          + jnp.dot(hc, w1c_ref[...], preferred_element_type=jnp.float32)
          + jnp.dot(ea_ref[...], w1a_ref[...], preferred_element_type=jnp.float32)
          + radial * w1rad_ref[...]
          + b1_ref[...])
    x = _silu(t1)
    m = _silu(jnp.dot(x, w2_ref[...], preferred_element_type=jnp.float32)
              + b2_ref[...])
    c1 = _silu(jnp.dot(m, wc1_ref[...], preferred_element_type=jnp.float32)
               + bc1_ref[...])
    w = jnp.sum(c1 * wc2_ref[...], axis=1, keepdims=True)
    cu = cd * (w * lax.rsqrt(radial + 1e-8))

    eye8 = jnp.eye(8, dtype=jnp.float32)
    rmT = lax.dot_general(rm_ref[...], eye8, (((0,), (0,)), ((), ())),
                          preferred_element_type=jnp.float32)
    lanebase = 4.0 * rmT[:, 0:1]
    lanes = lax.broadcasted_iota(jnp.int32, (BE, D), 1).astype(jnp.float32)
    cusp = (jnp.where(lanes == lanebase, cu[:, 0:1], 0.0)
            + jnp.where(lanes == lanebase + 1.0, cu[:, 1:2], 0.0)
            + jnp.where(lanes == lanebase + 2.0, cu[:, 2:3], 0.0)
            + jnp.where(lanes == lanebase + 3.0, 1.0, 0.0))
    out_ref[...] = jnp.concatenate([m, cusp], axis=1)


def _edge_mlp(gathered, edge_attr, rmsoa, w1r, w1c, w1a, w1rad, b1, w2, b2,
              wc1, bc1, wc2):
    nb = E // BE
    full = lambda shape: pl.BlockSpec(shape, lambda i: (0, 0))
    return pl.pallas_call(
        _edge_body,
        grid=(nb,),
        in_specs=[
            pl.BlockSpec((BE, DP), lambda i: (i, 0)),
            pl.BlockSpec((BE, DP), lambda i: (i + nb, 0)),
            pl.BlockSpec((BE, DE), lambda i: (i, 0)),
            pl.BlockSpec((8, BE), lambda i: (0, i)),
            full((D, D)), full((D, D)), full((DE, D)), full((1, D)),
            full((1, D)), full((D, D)), full((1, D)),
            full((D, D)), full((1, D)), full((1, D)),
        ],
        out_specs=pl.BlockSpec((BE, DP), lambda i: (i, 0)),
        out_shape=jax.ShapeDtypeStruct((E, DP), jnp.float32),
    )(gathered, gathered, edge_attr, rmsoa, w1r, w1c, w1a, w1rad, b1, w2, b2,
      wc1, bc1, wc2)


BN = 2000


def _node_body(p0_ref, p1_ref, q0_ref, q1_ref, h_ref, pos_ref, wn1h_ref,
               wn1m_ref, bn1_ref, wn2_ref, bn2_ref, hnew_ref, posnew_ref):
    m_i = p0_ref[...] + p1_ref[...]
    q = q0_ref[...] + q1_ref[...]
    pu = q[:, 0:3]
    cnt = q[:, 3:4]
    h = h_ref[...]
    t = _silu(jnp.dot(h, wn1h_ref[...], preferred_element_type=jnp.float32)
              + jnp.dot(m_i, wn1m_ref[...], preferred_element_type=jnp.float32)
              + bn1_ref[...])
    hnew_ref[...] = h + jnp.dot(t, wn2_ref[...], preferred_element_type=jnp.float32) + bn2_ref[...]
    posnew_ref[...] = pos_ref[...] + pu / (cnt + 1e-6)


def _node_mlp(p0, p1, q0, q1, h, pos, wn1h, wn1m, bn1, wn2, bn2):
    nb = N // BN
    full = lambda shape: pl.BlockSpec(shape, lambda i: (0, 0))
    return pl.pallas_call(
        _node_body,
        grid=(nb,),
        in_specs=[
            pl.BlockSpec((BN, D), lambda i: (i, 0)),
            pl.BlockSpec((BN, D), lambda i: (i, 0)),
            pl.BlockSpec((BN, 4), lambda i: (i, 0)),
            pl.BlockSpec((BN, 4), lambda i: (i, 0)),
            pl.BlockSpec((BN, D), lambda i: (i, 0)),
            pl.BlockSpec((BN, 3), lambda i: (i, 0)),
            full((D, D)), full((D, D)), full((1, D)),
            full((D, D)), full((1, D)),
        ],
        out_specs=[
            pl.BlockSpec((BN, D), lambda i: (i, 0)),
            pl.BlockSpec((BN, 3), lambda i: (i, 0)),
        ],
        out_shape=[
            jax.ShapeDtypeStruct((N, D), jnp.float32),
            jax.ShapeDtypeStruct((N, 3), jnp.float32),
        ],
    )(p0, p1, q0, q1, h, pos, wn1h, wn1m, bn1, wn2, bn2)


def kernel(h, pos, edge_index, edge_attr, W_e1, b_e1, W_e2, b_e2,
           W_n1, b_n1, W_n2, b_n2, W_c1, b_c1, W_c2):
    row, col = edge_index[0], edge_index[1]
    table = jnp.concatenate(
        [h, pos, jnp.zeros((N, DP - D - 3), jnp.float32)], axis=1)
    idx2 = jnp.concatenate([row, col])
    cuidx = lax.shift_right_logical(row, 5)
    rmod = jnp.remainder(row, 32).astype(jnp.float32)
    rmsoa = jnp.concatenate(
        [rmod.reshape(1, E), jnp.zeros((7, E), jnp.float32)], axis=0)

    gathered = _sc_gather(table, idx2)

    edgeout = _edge_mlp(gathered, edge_attr, rmsoa,
                        W_e1[:D], W_e1[D:2 * D], W_e1[2 * D + 1:],
                        W_e1[2 * D:2 * D + 1], b_e1.reshape(1, D),
                        W_e2, b_e2.reshape(1, D),
                        W_c1, b_c1.reshape(1, D), W_c2.reshape(1, D))

    outm, outcu = _sc_scatter(edgeout, row, cuidx)

    q0 = outcu[0].reshape(CUN * 32, 4)[:N]
    q1 = outcu[1].reshape(CUN * 32, 4)[:N]
    h_new, pos_new = _node_mlp(outm[0], outm[1], q0, q1, h, pos,
                               W_n1[:D], W_n1[D:], b_n1.reshape(1, D),
                               W_n2, b_n2.reshape(1, D))
    return (h_new, pos_new)


# speedup vs baseline: 3.4938x; 1.5856x over previous
import functools

import jax
import jax.numpy as jnp
from jax import lax
from jax.experimental import pallas as pl
from jax.experimental.pallas import tpu as pltpu
from jax.experimental.pallas import tpu_sc as plsc

N, E, D, DE = 10000, 320000, 128, 16
CUN = 320
NC, NS = 2, 16
NW = NC * NS
L = 16

GCH = 128
GCHUNKS_ALL = (2 * E) // GCH
GCHUNKS = -(-GCHUNKS_ALL // NW)

SCH = 128
SCHUNKS_ALL = E // SCH
SCHUNKS = -(-SCHUNKS_ALL // NW)
ZCH = 80
ZCHUNKS = N // ZCH
CUCHUNKS = CUN // ZCH


def _vector_mesh():
    return plsc.VectorSubcoreMesh(core_axis_name="c", subcore_axis_name="s")


_SC_PARAMS = pltpu.CompilerParams(needs_layout_passes=False)


@jax.jit
def _sc_gather(table, idx2, px, py, pz, row, col):
    @functools.partial(
        pl.kernel,
        mesh=_vector_mesh(),
        compiler_params=_SC_PARAMS,
        out_type=[
            jax.ShapeDtypeStruct((2 * E, D), jnp.float32),
            jax.ShapeDtypeStruct((8, E), jnp.float32),
        ],
        scratch_types=[
            pltpu.VMEM((GCH,), jnp.int32),
            pltpu.VMEM((GCH, D), jnp.float32),
            pltpu.VMEM((N,), jnp.float32),
            pltpu.VMEM((N,), jnp.float32),
            pltpu.VMEM((N,), jnp.float32),
            pltpu.VMEM((SCH,), jnp.int32),
            pltpu.VMEM((SCH,), jnp.int32),
            pltpu.VMEM((8, SCH), jnp.float32),
            pltpu.SemaphoreType.DMA,
        ],
    )
    def gk(table_hbm, idx_hbm, px_hbm, py_hbm, pz_hbm, row_hbm, col_hbm,
           out_hbm, cdr_hbm,
           idx_v, rows_v, px_v, py_v, pz_v, r_v, c_v, geo_v, sem):
        wid = lax.axis_index("c") * NS + lax.axis_index("s")

        pltpu.sync_copy(px_hbm, px_v)
        pltpu.sync_copy(py_hbm, py_v)
        pltpu.sync_copy(pz_hbm, pz_v)

        zero16 = jnp.zeros((L,), jnp.float32)

        @pl.loop(5, 8)
        def _(r):
            @pl.loop(0, SCH // L)
            def _(cc):
                geo_v[r, pl.ds(cc * L, L)] = zero16

        @pl.loop(0, SCHUNKS)
        def _(ch):
            cid = wid + ch * NW

            @pl.when(cid < SCHUNKS_ALL)
            def _():
                off = cid * SCH
                pltpu.sync_copy(row_hbm.at[pl.ds(off, SCH)], r_v)
                pltpu.sync_copy(col_hbm.at[pl.ds(off, SCH)], c_v)

                @pl.loop(0, SCH // L)
                def _(k):
                    sl = pl.ds(k * L, L)
                    ir = r_v[sl]
                    ic = c_v[sl]
                    dx = (plsc.load_gather(px_v, [ir])
                          - plsc.load_gather(px_v, [ic]))
                    dy = (plsc.load_gather(py_v, [ir])
                          - plsc.load_gather(py_v, [ic]))
                    dz = (plsc.load_gather(pz_v, [ir])
                          - plsc.load_gather(pz_v, [ic]))
                    geo_v[0, sl] = dx
                    geo_v[1, sl] = dy
                    geo_v[2, sl] = dz
                    geo_v[3, sl] = dx * dx + dy * dy + dz * dz
                    geo_v[4, sl] = lax.convert_element_type(
                        lax.bitwise_and(ir, 31), jnp.float32)

                pltpu.sync_copy(geo_v, cdr_hbm.at[:, pl.ds(off, SCH)])

        @pl.loop(0, GCHUNKS)
        def _(ch):
            cid = wid + ch * NW

            @pl.when(cid < GCHUNKS_ALL)
            def _():
                off = cid * GCH
                pltpu.sync_copy(idx_hbm.at[pl.ds(off, GCH)], idx_v)
                pltpu.async_copy(table_hbm.at[idx_v], rows_v, sem).wait()
                pltpu.sync_copy(rows_v, out_hbm.at[pl.ds(off, GCH)])

    return gk(table, idx2, px, py, pz, row, col)


@jax.jit
def _sc_scatter(mvals, cus, row):
    @functools.partial(
        pl.kernel,
        mesh=_vector_mesh(),
        compiler_params=_SC_PARAMS,
        out_type=[
            jax.ShapeDtypeStruct((NC, N, D), jnp.float32),
            jax.ShapeDtypeStruct((NC, CUN, D), jnp.float32),
        ],
        scratch_types=[
            pltpu.VMEM((SCH,), jnp.int32),
            pltpu.VMEM((SCH,), jnp.int32),
            pltpu.VMEM((SCH, D), jnp.float32),
            pltpu.VMEM((SCH, D), jnp.float32),
            pltpu.VMEM((8, SCH), jnp.float32),
            pltpu.VMEM_SHARED((N, D), jnp.float32),
            pltpu.VMEM_SHARED((CUN, D), jnp.float32),
            pltpu.SemaphoreType.DMA,
        ],
    )
    def sk(mvals_hbm, cus_hbm, idx_hbm, outm_hbm, outcu_hbm,
           idx_v, cuidx_v, mv, cuv, cus_v, macc, cuacc, sem):
        c = lax.axis_index("c")
        s = lax.axis_index("s")
        wid = c * NS + s

        zero16 = jnp.zeros((L,), jnp.float32)
        one16 = jnp.ones((L,), jnp.float32)

        @pl.loop(0, SCH)
        def _(r):
            @pl.loop(0, D // L)
            def _(cc):
                mv[r, pl.ds(cc * L, L)] = zero16
                cuv[r, pl.ds(cc * L, L)] = zero16

        @pl.loop(0, ZCHUNKS)
        def _(z):
            @pl.when(lax.rem(z, NS) == s)
            def _():
                pltpu.sync_copy(mv.at[pl.ds(0, ZCH)],
                                macc.at[pl.ds(z * ZCH, ZCH)])

        @pl.loop(0, CUCHUNKS)
        def _(z):
            @pl.when(z == s)
            def _():
                pltpu.sync_copy(mv.at[pl.ds(0, ZCH)],
                                cuacc.at[pl.ds(z * ZCH, ZCH)])

        plsc.subcore_barrier()

        lanes16 = lax.iota(jnp.int32, L)

        @pl.loop(0, SCHUNKS)
        def _(ch):
            cid = wid + ch * NW

            @pl.when(cid < SCHUNKS_ALL)
            def _():
                off = cid * SCH
                pltpu.sync_copy(idx_hbm.at[pl.ds(off, SCH)], idx_v)
                pltpu.sync_copy(mvals_hbm.at[pl.ds(off, SCH)], mv)
                pltpu.sync_copy(cus_hbm.at[:, pl.ds(off, SCH)], cus_v)

                @pl.loop(0, SCH // L)
                def _(k):
                    sl = pl.ds(k * L, L)
                    r16 = idx_v[sl]
                    rowi = lanes16 + k * L
                    lane = lax.shift_left(lax.bitwise_and(r16, 31), 2)
                    plsc.store_scatter(cuv, [rowi, lane], cus_v[0, sl])
                    plsc.store_scatter(cuv, [rowi, lane + 1], cus_v[1, sl])
                    plsc.store_scatter(cuv, [rowi, lane + 2], cus_v[2, sl])
                    plsc.store_scatter(cuv, [rowi, lane + 3], one16)
                    cuidx_v[sl] = lax.shift_right_logical(r16, 5)

                pltpu.sync_copy(mv, macc.at[idx_v], add=True)
                pltpu.sync_copy(cuv, cuacc.at[cuidx_v], add=True)

                @pl.loop(0, SCH // L)
                def _(k):
                    sl = pl.ds(k * L, L)
                    r16 = idx_v[sl]
                    rowi = lanes16 + k * L
                    lane = lax.shift_left(lax.bitwise_and(r16, 31), 2)
                    plsc.store_scatter(cuv, [rowi, lane], zero16)
                    plsc.store_scatter(cuv, [rowi, lane + 1], zero16)
                    plsc.store_scatter(cuv, [rowi, lane + 2], zero16)
                    plsc.store_scatter(cuv, [rowi, lane + 3], zero16)

        plsc.subcore_barrier()

        @pl.loop(0, ZCHUNKS)
        def _(z):
            @pl.when(lax.rem(z, NS) == s)
            def _():
                pltpu.sync_copy(macc.at[pl.ds(z * ZCH, ZCH)],
                                outm_hbm.at[c, pl.ds(z * ZCH, ZCH)])

        @pl.loop(0, CUCHUNKS)
        def _(z):
            @pl.when(z == s)
            def _():
                pltpu.sync_copy(cuacc.at[pl.ds(z * ZCH, ZCH)],
                                outcu_hbm.at[c, pl.ds(z * ZCH, ZCH)])

    return sk(mvals, cus, row)


BE = 2560


def _silu(x):
    return x * jax.nn.sigmoid(x)


def _bf(x):
    return x.astype(jnp.bfloat16)


def _edge_body(gr_ref, gc_ref, ea_ref, cdr_ref, w1r_ref, w1c_ref, w1a_ref,
               w1rad_ref, b1_ref, w2_ref, b2_ref, wc1_ref, bc1_ref, wc2_ref,
               m_ref, cus_ref):
    eye8 = jnp.eye(8, dtype=jnp.float32)
    cdrT = lax.dot_general(cdr_ref[...], eye8, (((0,), (0,)), ((), ())),
                           preferred_element_type=jnp.float32)
    cd = cdrT[:, 0:3]
    radial = cdrT[:, 3:4]

    t1 = (jnp.dot(_bf(gr_ref[...]), w1r_ref[...],
                  preferred_element_type=jnp.float32)
          + jnp.dot(_bf(gc_ref[...]), w1c_ref[...],
                    preferred_element_type=jnp.float32)
          + jnp.dot(_bf(ea_ref[...]), w1a_ref[...],
                    preferred_element_type=jnp.float32)
          + radial * w1rad_ref[...]
          + b1_ref[...])
    x = _silu(t1)
    m = _silu(jnp.dot(_bf(x), w2_ref[...], preferred_element_type=jnp.float32)
              + b2_ref[...])
    c1 = _silu(jnp.dot(_bf(m), wc1_ref[...], preferred_element_type=jnp.float32)
               + bc1_ref[...])
    w = jnp.sum(c1 * wc2_ref[...], axis=1, keepdims=True)
    cu = cd * (w * lax.rsqrt(radial + 1e-8))

    m_ref[...] = m
    cuT = jnp.concatenate([cu, jnp.zeros((BE, 5), jnp.float32)], axis=1)
    cus_ref[...] = lax.dot_general(eye8, cuT, (((1,), (1,)), ((), ())),
                                   preferred_element_type=jnp.float32)


def _edge_mlp(gathered, edge_attr, cdr, w1r, w1c, w1a, w1rad, b1, w2, b2,
              wc1, bc1, wc2):
    nb = E // BE
    full = lambda shape: pl.BlockSpec(shape, lambda i: (0, 0))
    return pl.pallas_call(
        _edge_body,
        grid=(nb,),
        in_specs=[
            pl.BlockSpec((BE, D), lambda i: (i, 0)),
            pl.BlockSpec((BE, D), lambda i: (i + nb, 0)),
            pl.BlockSpec((BE, DE), lambda i: (i, 0)),
            pl.BlockSpec((8, BE), lambda i: (0, i)),
            full((D, D)), full((D, D)), full((DE, D)), full((1, D)),
            full((1, D)), full((D, D)), full((1, D)),
            full((D, D)), full((1, D)), full((1, D)),
        ],
        out_specs=[
            pl.BlockSpec((BE, D), lambda i: (i, 0)),
            pl.BlockSpec((8, BE), lambda i: (0, i)),
        ],
        out_shape=[
            jax.ShapeDtypeStruct((E, D), jnp.float32),
            jax.ShapeDtypeStruct((8, E), jnp.float32),
        ],
    )(gathered, gathered, edge_attr, cdr, w1r, w1c, w1a, w1rad, b1, w2, b2,
      wc1, bc1, wc2)


BN = 2000


def _node_body(p0_ref, p1_ref, q0_ref, q1_ref, h_ref, pos_ref, wn1h_ref,
               wn1m_ref, bn1_ref, wn2_ref, bn2_ref, hnew_ref, posnew_ref):
    m_i = p0_ref[...] + p1_ref[...]
    q = q0_ref[...] + q1_ref[...]
    pu = q[:, 0:3]
    cnt = q[:, 3:4]
    h = h_ref[...]
    t = _silu(jnp.dot(_bf(h), wn1h_ref[...], preferred_element_type=jnp.float32)
              + jnp.dot(_bf(m_i), wn1m_ref[...],
                        preferred_element_type=jnp.float32)
              + bn1_ref[...])
    hnew_ref[...] = h + jnp.dot(_bf(t), wn2_ref[...],
                                preferred_element_type=jnp.float32) + bn2_ref[...]
    posnew_ref[...] = pos_ref[...] + pu / (cnt + 1e-6)


def _node_mlp(p0, p1, q0, q1, h, pos, wn1h, wn1m, bn1, wn2, bn2):
    nb = N // BN
    full = lambda shape: pl.BlockSpec(shape, lambda i: (0, 0))
    return pl.pallas_call(
        _node_body,
        grid=(nb,),
        in_specs=[
            pl.BlockSpec((BN, D), lambda i: (i, 0)),
            pl.BlockSpec((BN, D), lambda i: (i, 0)),
            pl.BlockSpec((BN, 4), lambda i: (i, 0)),
            pl.BlockSpec((BN, 4), lambda i: (i, 0)),
            pl.BlockSpec((BN, D), lambda i: (i, 0)),
            pl.BlockSpec((BN, 3), lambda i: (i, 0)),
            full((D, D)), full((D, D)), full((1, D)),
            full((D, D)), full((1, D)),
        ],
        out_specs=[
            pl.BlockSpec((BN, D), lambda i: (i, 0)),
            pl.BlockSpec((BN, 3), lambda i: (i, 0)),
        ],
        out_shape=[
            jax.ShapeDtypeStruct((N, D), jnp.float32),
            jax.ShapeDtypeStruct((N, 3), jnp.float32),
        ],
    )(p0, p1, q0, q1, h, pos, wn1h, wn1m, bn1, wn2, bn2)


def kernel(h, pos, edge_index, edge_attr, W_e1, b_e1, W_e2, b_e2,
           W_n1, b_n1, W_n2, b_n2, W_c1, b_c1, W_c2):
    row, col = edge_index[0], edge_index[1]
    idx2 = jnp.concatenate([row, col])

    gathered, cdr = _sc_gather(h, idx2, pos[:, 0], pos[:, 1], pos[:, 2],
                               row, col)

    bf = lambda w: w.astype(jnp.bfloat16)
    mvals, cus = _edge_mlp(gathered, edge_attr, cdr,
                           bf(W_e1[:D]), bf(W_e1[D:2 * D]),
                           bf(W_e1[2 * D + 1:]), W_e1[2 * D:2 * D + 1],
                           b_e1.reshape(1, D),
                           bf(W_e2), b_e2.reshape(1, D),
                           bf(W_c1), b_c1.reshape(1, D), W_c2.reshape(1, D))

    outm, outcu = _sc_scatter(mvals, cus, row)

    q0 = outcu[0].reshape(CUN * 32, 4)[:N]
    q1 = outcu[1].reshape(CUN * 32, 4)[:N]
    h_new, pos_new = _node_mlp(outm[0], outm[1], q0, q1, h, pos,
                               bf(W_n1[:D]), bf(W_n1[D:]), b_n1.reshape(1, D),
                               bf(W_n2), b_n2.reshape(1, D))
    return (h_new, pos_new)


# speedup vs baseline: 4.5269x; 1.2957x over previous
import functools

import jax
import jax.numpy as jnp
from jax import lax
from jax.experimental import pallas as pl
from jax.experimental.pallas import tpu as pltpu
from jax.experimental.pallas import tpu_sc as plsc

N, E, D, DE = 10000, 320000, 128, 16
CUN = 320
NC, NS = 2, 16
NW = NC * NS
L = 16
CH = 128
ZCH = 80
ZCHUNKS = N // ZCH
CUCHUNKS = CUN // ZCH
NSLICE = 4
ESL = E // NSLICE


def _vector_mesh():
    return plsc.VectorSubcoreMesh(core_axis_name="c", subcore_axis_name="s")


_SC_PARAMS = pltpu.CompilerParams(needs_layout_passes=False)


@jax.jit
def _sc_gather(table, idx2, px, py, pz, row, col):
    ne = row.shape[0]
    gchunks_all = (2 * ne) // CH
    gchunks = -(-gchunks_all // NW)
    echunks_all = ne // CH
    echunks = -(-echunks_all // NW)

    @functools.partial(
        pl.kernel,
        mesh=_vector_mesh(),
        compiler_params=_SC_PARAMS,
        out_type=[
            jax.ShapeDtypeStruct((2 * ne, D), jnp.float32),
            jax.ShapeDtypeStruct((8, ne), jnp.float32),
        ],
        scratch_types=[
            pltpu.VMEM((CH,), jnp.int32),
            pltpu.VMEM((CH, D), jnp.float32),
            pltpu.VMEM((N,), jnp.float32),
            pltpu.VMEM((N,), jnp.float32),
            pltpu.VMEM((N,), jnp.float32),
            pltpu.VMEM((CH,), jnp.int32),
            pltpu.VMEM((CH,), jnp.int32),
            pltpu.VMEM((8, CH), jnp.float32),
            pltpu.SemaphoreType.DMA,
        ],
    )
    def gk(table_hbm, idx_hbm, px_hbm, py_hbm, pz_hbm, row_hbm, col_hbm,
           out_hbm, cdr_hbm,
           idx_v, rows_v, px_v, py_v, pz_v, r_v, c_v, geo_v, sem):
        wid = lax.axis_index("c") * NS + lax.axis_index("s")

        pltpu.sync_copy(px_hbm, px_v)
        pltpu.sync_copy(py_hbm, py_v)
        pltpu.sync_copy(pz_hbm, pz_v)

        zero16 = jnp.zeros((L,), jnp.float32)

        @pl.loop(5, 8)
        def _(r):
            @pl.loop(0, CH // L)
            def _(cc):
                geo_v[r, pl.ds(cc * L, L)] = zero16

        @pl.loop(0, echunks)
        def _(ch):
            cid = wid + ch * NW

            @pl.when(cid < echunks_all)
            def _():
                off = cid * CH
                pltpu.sync_copy(row_hbm.at[pl.ds(off, CH)], r_v)
                pltpu.sync_copy(col_hbm.at[pl.ds(off, CH)], c_v)

                @pl.loop(0, CH // L)
                def _(k):
                    sl = pl.ds(k * L, L)
                    ir = r_v[sl]
                    ic = c_v[sl]
                    dx = (plsc.load_gather(px_v, [ir])
                          - plsc.load_gather(px_v, [ic]))
                    dy = (plsc.load_gather(py_v, [ir])
                          - plsc.load_gather(py_v, [ic]))
                    dz = (plsc.load_gather(pz_v, [ir])
                          - plsc.load_gather(pz_v, [ic]))
                    geo_v[0, sl] = dx
                    geo_v[1, sl] = dy
                    geo_v[2, sl] = dz
                    geo_v[3, sl] = dx * dx + dy * dy + dz * dz
                    geo_v[4, sl] = lax.convert_element_type(
                        lax.bitwise_and(ir, 31), jnp.float32)

                pltpu.sync_copy(geo_v, cdr_hbm.at[:, pl.ds(off, CH)])

        @pl.loop(0, gchunks)
        def _(ch):
            cid = wid + ch * NW

            @pl.when(cid < gchunks_all)
            def _():
                off = cid * CH
                pltpu.sync_copy(idx_hbm.at[pl.ds(off, CH)], idx_v)
                pltpu.async_copy(table_hbm.at[idx_v], rows_v, sem).wait()
                pltpu.sync_copy(rows_v, out_hbm.at[pl.ds(off, CH)])

    return gk(table, idx2, px, py, pz, row, col)


@jax.jit
def _sc_scatter(mvals, cus, row):
    ne = row.shape[0]
    echunks_all = ne // CH
    echunks = -(-echunks_all // NW)

    @functools.partial(
        pl.kernel,
        mesh=_vector_mesh(),
        compiler_params=_SC_PARAMS,
        out_type=[
            jax.ShapeDtypeStruct((NC, N, D), jnp.float32),
            jax.ShapeDtypeStruct((NC, CUN, D), jnp.float32),
        ],
        scratch_types=[
            pltpu.VMEM((CH,), jnp.int32),
            pltpu.VMEM((CH,), jnp.int32),
            pltpu.VMEM((CH, D), jnp.float32),
            pltpu.VMEM((CH, D), jnp.float32),
            pltpu.VMEM((8, CH), jnp.float32),
            pltpu.VMEM_SHARED((N, D), jnp.float32),
            pltpu.VMEM_SHARED((CUN, D), jnp.float32),
            pltpu.SemaphoreType.DMA,
        ],
    )
    def sk(mvals_hbm, cus_hbm, idx_hbm, outm_hbm, outcu_hbm,
           idx_v, cuidx_v, mv, cuv, cus_v, macc, cuacc, sem):
        c = lax.axis_index("c")
        s = lax.axis_index("s")
        wid = c * NS + s

        zero16 = jnp.zeros((L,), jnp.float32)
        one16 = jnp.ones((L,), jnp.float32)

        @pl.loop(0, CH)
        def _(r):
            @pl.loop(0, D // L)
            def _(cc):
                mv[r, pl.ds(cc * L, L)] = zero16
                cuv[r, pl.ds(cc * L, L)] = zero16

        @pl.loop(0, ZCHUNKS)
        def _(z):
            @pl.when(lax.rem(z, NS) == s)
            def _():
                pltpu.sync_copy(mv.at[pl.ds(0, ZCH)],
                                macc.at[pl.ds(z * ZCH, ZCH)])

        @pl.loop(0, CUCHUNKS)
        def _(z):
            @pl.when(z == s)
            def _():
                pltpu.sync_copy(mv.at[pl.ds(0, ZCH)],
                                cuacc.at[pl.ds(z * ZCH, ZCH)])

        plsc.subcore_barrier()

        lanes16 = lax.iota(jnp.int32, L)

        @pl.loop(0, echunks)
        def _(ch):
            cid = wid + ch * NW

            @pl.when(cid < echunks_all)
            def _():
                off = cid * CH
                pltpu.sync_copy(idx_hbm.at[pl.ds(off, CH)], idx_v)
                pltpu.sync_copy(mvals_hbm.at[pl.ds(off, CH)], mv)
                pltpu.sync_copy(cus_hbm.at[:, pl.ds(off, CH)], cus_v)

                @pl.loop(0, CH // L)
                def _(k):
                    sl = pl.ds(k * L, L)
                    r16 = idx_v[sl]
                    rowi = lanes16 + k * L
                    lane = lax.shift_left(lax.bitwise_and(r16, 31), 2)
                    plsc.store_scatter(cuv, [rowi, lane], cus_v[0, sl])
                    plsc.store_scatter(cuv, [rowi, lane + 1], cus_v[1, sl])
                    plsc.store_scatter(cuv, [rowi, lane + 2], cus_v[2, sl])
                    plsc.store_scatter(cuv, [rowi, lane + 3], one16)
                    cuidx_v[sl] = lax.shift_right_logical(r16, 5)

                pltpu.sync_copy(mv, macc.at[idx_v], add=True)
                pltpu.sync_copy(cuv, cuacc.at[cuidx_v], add=True)

                @pl.loop(0, CH // L)
                def _(k):
                    sl = pl.ds(k * L, L)
                    r16 = idx_v[sl]
                    rowi = lanes16 + k * L
                    lane = lax.shift_left(lax.bitwise_and(r16, 31), 2)
                    plsc.store_scatter(cuv, [rowi, lane], zero16)
                    plsc.store_scatter(cuv, [rowi, lane + 1], zero16)
                    plsc.store_scatter(cuv, [rowi, lane + 2], zero16)
                    plsc.store_scatter(cuv, [rowi, lane + 3], zero16)

        plsc.subcore_barrier()

        @pl.loop(0, ZCHUNKS)
        def _(z):
            @pl.when(lax.rem(z, NS) == s)
            def _():
                pltpu.sync_copy(macc.at[pl.ds(z * ZCH, ZCH)],
                                outm_hbm.at[c, pl.ds(z * ZCH, ZCH)])

        @pl.loop(0, CUCHUNKS)
        def _(z):
            @pl.when(z == s)
            def _():
                pltpu.sync_copy(cuacc.at[pl.ds(z * ZCH, ZCH)],
                                outcu_hbm.at[c, pl.ds(z * ZCH, ZCH)])

    return sk(mvals, cus, row)


BE = 3200


def _silu(x):
    return x * jax.nn.sigmoid(x)


def _bf(x):
    return x.astype(jnp.bfloat16)


def _edge_body(gr_ref, gc_ref, ea_ref, cdr_ref, w1r_ref, w1c_ref, w1a_ref,
               w1rad_ref, b1_ref, w2_ref, b2_ref, wc1_ref, bc1_ref, wc2_ref,
               m_ref, cus_ref):
    eye8 = jnp.eye(8, dtype=jnp.float32)
    cdrT = lax.dot_general(cdr_ref[...], eye8, (((0,), (0,)), ((), ())),
                           preferred_element_type=jnp.float32)
    cd = cdrT[:, 0:3]
    radial = cdrT[:, 3:4]

    t1 = (jnp.dot(_bf(gr_ref[...]), w1r_ref[...],
                  preferred_element_type=jnp.float32)
          + jnp.dot(_bf(gc_ref[...]), w1c_ref[...],
                    preferred_element_type=jnp.float32)
          + jnp.dot(_bf(ea_ref[...]), w1a_ref[...],
                    preferred_element_type=jnp.float32)
          + radial * w1rad_ref[...]
          + b1_ref[...])
    x = _silu(t1)
    m = _silu(jnp.dot(_bf(x), w2_ref[...], preferred_element_type=jnp.float32)
              + b2_ref[...])
    c1 = _silu(jnp.dot(_bf(m), wc1_ref[...], preferred_element_type=jnp.float32)
               + bc1_ref[...])
    w = jnp.sum(c1 * wc2_ref[...], axis=1, keepdims=True)
    cu = cd * (w * lax.rsqrt(radial + 1e-8))

    m_ref[...] = m
    cuT = jnp.concatenate([cu, jnp.zeros((BE, 5), jnp.float32)], axis=1)
    cus_ref[...] = lax.dot_general(eye8, cuT, (((1,), (1,)), ((), ())),
                                   preferred_element_type=jnp.float32)


def _edge_mlp(gathered, edge_attr, cdr, w1r, w1c, w1a, w1rad, b1, w2, b2,
              wc1, bc1, wc2):
    ne = edge_attr.shape[0]
    nb = ne // BE
    full = lambda shape: pl.BlockSpec(shape, lambda i: (0, 0))
    return pl.pallas_call(
        _edge_body,
        grid=(nb,),
        in_specs=[
            pl.BlockSpec((BE, D), lambda i: (i, 0)),
            pl.BlockSpec((BE, D), lambda i: (i + nb, 0)),
            pl.BlockSpec((BE, DE), lambda i: (i, 0)),
            pl.BlockSpec((8, BE), lambda i: (0, i)),
            full((D, D)), full((D, D)), full((DE, D)), full((1, D)),
            full((1, D)), full((D, D)), full((1, D)),
            full((D, D)), full((1, D)), full((1, D)),
        ],
        out_specs=[
            pl.BlockSpec((BE, D), lambda i: (i, 0)),
            pl.BlockSpec((8, BE), lambda i: (0, i)),
        ],
        out_shape=[
            jax.ShapeDtypeStruct((ne, D), jnp.float32),
            jax.ShapeDtypeStruct((8, ne), jnp.float32),
        ],
    )(gathered, gathered, edge_attr, cdr, w1r, w1c, w1a, w1rad, b1, w2, b2,
      wc1, bc1, wc2)


BN = 2000


def _node_body(p_ref, q_ref, h_ref, pos_ref, wn1h_ref, wn1m_ref, bn1_ref,
               wn2_ref, bn2_ref, hnew_ref, posnew_ref):
    m_i = p_ref[...]
    q = q_ref[...]
    pu = q[:, 0:3]
    cnt = q[:, 3:4]
    h = h_ref[...]
    t = _silu(jnp.dot(_bf(h), wn1h_ref[...], preferred_element_type=jnp.float32)
              + jnp.dot(_bf(m_i), wn1m_ref[...],
                        preferred_element_type=jnp.float32)
              + bn1_ref[...])
    hnew_ref[...] = h + jnp.dot(_bf(t), wn2_ref[...],
                                preferred_element_type=jnp.float32) + bn2_ref[...]
    posnew_ref[...] = pos_ref[...] + pu / (cnt + 1e-6)


def _node_mlp(p, q, h, pos, wn1h, wn1m, bn1, wn2, bn2):
    nb = N // BN
    full = lambda shape: pl.BlockSpec(shape, lambda i: (0, 0))
    return pl.pallas_call(
        _node_body,
        grid=(nb,),
        in_specs=[
            pl.BlockSpec((BN, D), lambda i: (i, 0)),
            pl.BlockSpec((BN, 4), lambda i: (i, 0)),
            pl.BlockSpec((BN, D), lambda i: (i, 0)),
            pl.BlockSpec((BN, 3), lambda i: (i, 0)),
            full((D, D)), full((D, D)), full((1, D)),
            full((D, D)), full((1, D)),
        ],
        out_specs=[
            pl.BlockSpec((BN, D), lambda i: (i, 0)),
            pl.BlockSpec((BN, 3), lambda i: (i, 0)),
        ],
        out_shape=[
            jax.ShapeDtypeStruct((N, D), jnp.float32),
            jax.ShapeDtypeStruct((N, 3), jnp.float32),
        ],
    )(p, q, h, pos, wn1h, wn1m, bn1, wn2, bn2)


def kernel(h, pos, edge_index, edge_attr, W_e1, b_e1, W_e2, b_e2,
           W_n1, b_n1, W_n2, b_n2, W_c1, b_c1, W_c2):
    row, col = edge_index[0], edge_index[1]
    px, py, pz = pos[:, 0], pos[:, 1], pos[:, 2]

    bf = lambda w: w.astype(jnp.bfloat16)
    ew = (bf(W_e1[:D]), bf(W_e1[D:2 * D]), bf(W_e1[2 * D + 1:]),
          W_e1[2 * D:2 * D + 1], b_e1.reshape(1, D),
          bf(W_e2), b_e2.reshape(1, D),
          bf(W_c1), b_c1.reshape(1, D), W_c2.reshape(1, D))

    outms, outcus = [], []
    for k in range(NSLICE):
        sl = slice(k * ESL, (k + 1) * ESL)
        row_k, col_k = row[sl], col[sl]
        idx2_k = jnp.concatenate([row_k, col_k])
        gathered, cdr = _sc_gather(h, idx2_k, px, py, pz, row_k, col_k)
        mvals, cus = _edge_mlp(gathered, edge_attr[sl], cdr, *ew)
        outm, outcu = _sc_scatter(mvals, cus, row_k)
        outms.append(outm)
        outcus.append(outcu)

    m_i = sum(o[0] + o[1] for o in outms)
    qacc = sum(o[0] + o[1] for o in outcus)
    q = qacc.reshape(CUN * 32, 4)[:N]
    h_new, pos_new = _node_mlp(m_i, q, h, pos,
                               bf(W_n1[:D]), bf(W_n1[D:]), b_n1.reshape(1, D),
                               bf(W_n2), b_n2.reshape(1, D))
    return (h_new, pos_new)
